# baseline (speedup 1.0000x reference)
import functools

import jax
import jax.numpy as jnp
from jax import lax
from jax.experimental import pallas as pl
from jax.experimental.pallas import tpu as pltpu
from jax.experimental.pallas import tpu_sc as plsc

N = 10000
E = 160000
C = 20000
B = 64
H = 32
NF = 128
EF = 16
EH = 64
STEPS = 3

MSG_TILE = 1000


def _bn(x, g, b):
    m = jnp.mean(x, axis=0)
    v = jnp.var(x, axis=0)
    return g * (x - m) / jnp.sqrt(v + 1e-5) + b


def _msg_body(a_ref, e4_ref, w4t_ref, b4_ref, s_ref, out_ref):
    wt = jnp.dot(e4_ref[...], w4t_ref[...], preferred_element_type=jnp.float32)
    wt += b4_ref[0:1, :]
    a_rep = jnp.dot(a_ref[...], s_ref[...], preferred_element_type=jnp.float32)
    r = a_rep * wt
    r = r[:, :512] + r[:, 512:]
    r = r[:, :256] + r[:, 256:]
    r = r[:, :128] + r[:, 128:]
    r = r[:, :64] + r[:, 64:]
    out_ref[...] = r[:, :32] + r[:, 32:]


@functools.partial(jax.jit, static_argnames=("tile",))
def _msg_matmul(a_src, e4, w4t, b4row, sel, tile=MSG_TILE):
    return pl.pallas_call(
        _msg_body,
        grid=(E // tile,),
        in_specs=[
            pl.BlockSpec((tile, H), lambda i: (i, 0)),
            pl.BlockSpec((tile, EH), lambda i: (i, 0)),
            pl.BlockSpec((EH, H * H), lambda i: (0, 0)),
            pl.BlockSpec((8, H * H), lambda i: (0, 0)),
            pl.BlockSpec((H, H * H), lambda i: (0, 0)),
        ],
        out_specs=pl.BlockSpec((tile, H), lambda i: (i, 0)),
        out_shape=jax.ShapeDtypeStruct((E, H), jnp.float32),
    )(a_src, e4, w4t, b4row, sel)


def _lin_body(x_ref, w_ref, b_ref, y_ref, st_ref, *, relu):
    y = jnp.dot(x_ref[...], w_ref[...], preferred_element_type=jnp.float32) + b_ref[0:1, :]
    if relu:
        y = jnp.maximum(y, 0.0)
    y_ref[...] = y

    @pl.when(pl.program_id(0) == 0)
    def _():
        st_ref[...] = jnp.zeros_like(st_ref)
    st_ref[0:1, :] += jnp.sum(y, axis=0, keepdims=True)
    st_ref[1:2, :] += jnp.sum(y * y, axis=0, keepdims=True)


@functools.partial(jax.jit, static_argnames=("tile", "relu"))
def _lin_layer(x, wp, bp, tile, relu=True):
    rows, din = x.shape
    dout = wp.shape[1]
    return pl.pallas_call(
        functools.partial(_lin_body, relu=relu),
        grid=(rows // tile,),
        in_specs=[
            pl.BlockSpec((tile, din), lambda i: (i, 0)),
            pl.BlockSpec((din, dout), lambda i: (0, 0)),
            pl.BlockSpec((8, dout), lambda i: (0, 0)),
        ],
        out_specs=[
            pl.BlockSpec((tile, dout), lambda i: (i, 0)),
            pl.BlockSpec((8, dout), lambda i: (0, 0)),
        ],
        out_shape=[
            jax.ShapeDtypeStruct((rows, dout), jnp.float32),
            jax.ShapeDtypeStruct((8, dout), jnp.float32),
        ],
    )(x, wp, bp)


def _stats_body(x_ref, st_ref):
    x = x_ref[...]

    @pl.when(pl.program_id(0) == 0)
    def _():
        st_ref[...] = jnp.zeros_like(st_ref)
    st_ref[0:1, :] += jnp.sum(x, axis=0, keepdims=True)
    st_ref[1:2, :] += jnp.sum(x * x, axis=0, keepdims=True)


@functools.partial(jax.jit, static_argnames=("tile",))
def _col_stats(x, tile):
    rows, d = x.shape
    return pl.pallas_call(
        _stats_body,
        grid=(rows // tile,),
        in_specs=[pl.BlockSpec((tile, d), lambda i: (i, 0))],
        out_specs=pl.BlockSpec((8, d), lambda i: (0, 0)),
        out_shape=jax.ShapeDtypeStruct((8, d), jnp.float32),
    )(x)


def _fold(st, rows, g, b_bn, wT, b):
    m = st[0] / rows
    v = st[1] / rows - m * m
    sc = g / jnp.sqrt(v + 1e-5)
    sh = b_bn - m * sc
    wp = sc[:, None] * wT
    bp = jnp.broadcast_to((sh @ wT + b)[None, :], (8, wT.shape[1]))
    return wp, bp


_SC_NC = 2
_SC_NS = 16
_CH = 128
_NCHUNK = E // _CH
_CPC = _NCHUNK // _SC_NC
_STRIPE = 624
_TAIL = N - _STRIPE * _SC_NS


@jax.jit
def _sc_scatter_add(msg3, dst_flat, zstripe):
    mesh = plsc.VectorSubcoreMesh(core_axis_name="c", subcore_axis_name="s")

    @functools.partial(
        pl.kernel, mesh=mesh,
        out_type=jax.ShapeDtypeStruct((_SC_NC, N, H), jnp.float32),
        scratch_types=[
            pltpu.VMEM((_CH,), jnp.int32),
            pltpu.VMEM((_CH, H), jnp.float32),
            pltpu.VMEM_SHARED((N, H), jnp.float32),
        ],
        compiler_params=pltpu.CompilerParams(use_tc_tiling_on_sc=False),
    )
    def k(msg_hbm, dst_hbm, z_hbm, out_hbm, idx_v, rows_v, agg_sh):
        c = lax.axis_index("c")
        s = lax.axis_index("s")
        pltpu.sync_copy(z_hbm, agg_sh.at[pl.ds(s * _STRIPE, _STRIPE)])

        @pl.when(s == _SC_NS - 1)
        def _():
            pltpu.sync_copy(z_hbm.at[pl.ds(0, _TAIL)],
                            agg_sh.at[pl.ds(_STRIPE * _SC_NS, _TAIL)])
        plsc.subcore_barrier()

        def body(j, carry):
            cid_local = j * _SC_NS + s

            @pl.when(cid_local < _CPC)
            def _():
                cid = c * _CPC + cid_local
                pltpu.sync_copy(dst_hbm.at[pl.ds(cid * _CH, _CH)], idx_v)
                pltpu.sync_copy(msg_hbm.at[cid], rows_v)
                pltpu.sync_copy(rows_v, agg_sh.at[idx_v], add=True)
            return carry

        lax.fori_loop(0, (_CPC + _SC_NS - 1) // _SC_NS, body, 0)
        plsc.subcore_barrier()
        pltpu.sync_copy(agg_sh.at[pl.ds(s * _STRIPE, _STRIPE)],
                        out_hbm.at[c].at[pl.ds(s * _STRIPE, _STRIPE)])

        @pl.when(s == _SC_NS - 1)
        def _():
            pltpu.sync_copy(agg_sh.at[pl.ds(_STRIPE * _SC_NS, _TAIL)],
                            out_hbm.at[c].at[pl.ds(_STRIPE * _SC_NS, _TAIL)])

    return k(msg3, dst_flat, zstripe)


def _segment_sum_sc(msg, dst_flat, zstripe):
    parts = _sc_scatter_add(msg.reshape(_NCHUNK, _CH, H), dst_flat, zstripe)
    return parts[0] + parts[1]


def kernel(node, edge, edge_index, node_batch_index, coupling_index, coupling_type, coupling_type_back, coupling_value, coupling_batch_index, params):
    p = params
    ei = edge_index.T
    src = ei[0]
    dst = ei[1]
    stN = _col_stats(node, 1000)
    ew1, eb1 = _fold(stN, N, p['emb_bn1_g'], p['emb_bn1_b'], p['emb_W1'].T, p['emb_b1'])
    xe1, stE1 = _lin_layer(node, ew1, eb1, 1000)
    ew2, eb2 = _fold(stE1, N, p['emb_bn2_g'], p['emb_bn2_b'], p['emb_W2'].T,
                     jnp.zeros((H,), jnp.float32))
    x = _lin_layer(xe1, ew2, eb2, 1000)[0]
    h = x
    acc = x
    st0 = _col_stats(edge, 1000)
    w1p, b1p = _fold(st0, E, p['en_bn1_g'], p['en_bn1_b'], p['en_W1'].T, p['en_b1'])
    y1, st1 = _lin_layer(edge, w1p, b1p, 1000)
    w2p, b2p = _fold(st1, E, p['en_bn2_g'], p['en_bn2_b'], p['en_W2'].T, p['en_b2'])
    y2, st2 = _lin_layer(y1, w2p, b2p, 1000)
    w3p, b3p = _fold(st2, E, p['en_bn3_g'], p['en_bn3_b'], p['en_W3'].T, p['en_b3'])
    y3, st3 = _lin_layer(y2, w3p, b3p, 1000)
    m4 = st3[0] / E
    v4 = st3[1] / E - m4 * m4
    sc4 = p['en_bn4_g'] / jnp.sqrt(v4 + 1e-5)
    sh4 = p['en_bn4_b'] - m4 * sc4
    w4t = sc4[:, None] * p['en_W4'].T
    b4row = jnp.broadcast_to((p['en_b4'] + sh4 @ p['en_W4'].T)[None, :], (8, H * H))
    counts = jnp.maximum(jax.ops.segment_sum(jnp.ones(E, jnp.float32), dst, num_segments=N), 1.0)[:, None]
    sel = jnp.repeat(jnp.eye(H, dtype=jnp.float32), H, axis=1)
    dst_flat = dst.astype(jnp.int32)
    zstripe = jnp.zeros((_STRIPE, H), jnp.float32)
    for _ in range(STEPS):
        a_src = acc[src]
        msg = _msg_matmul(a_src, y3, w4t, b4row, sel)
        agg = _segment_sum_sc(msg, dst_flat, zstripe) / counts
        m = jax.nn.relu(agg + acc @ p['conv_root'].T + p['conv_bias'])
        gi = m @ p['gru_W_ih'].T + p['gru_b_ih']
        gh = h @ p['gru_W_hh'].T + p['gru_b_hh']
        i_r, i_z, i_n = jnp.split(gi, 3, axis=1)
        h_r, h_z, h_n = jnp.split(gh, 3, axis=1)
        r = jax.nn.sigmoid(i_r + h_r)
        z = jax.nn.sigmoid(i_z + h_z)
        n = jnp.tanh(i_n + r * h_n)
        acc = (1.0 - z) * n + z * h
        h = acc
    q_star = jnp.zeros((B, 2 * H), jnp.float32)
    hl = jnp.zeros((B, H), jnp.float32)
    cl = jnp.zeros((B, H), jnp.float32)
    for _ in range(STEPS):
        gates = q_star @ p['lstm_W_ih'].T + p['lstm_b_ih'] + hl @ p['lstm_W_hh'].T + p['lstm_b_hh']
        gi_, gf_, gg_, go_ = jnp.split(gates, 4, axis=1)
        cl = jax.nn.sigmoid(gf_) * cl + jax.nn.sigmoid(gi_) * jnp.tanh(gg_)
        hl = jax.nn.sigmoid(go_) * jnp.tanh(cl)
        eatt = jnp.sum(acc * hl[node_batch_index], axis=-1)
        emax = jax.ops.segment_max(eatt, node_batch_index, num_segments=B)
        a = jnp.exp(eatt - emax[node_batch_index])
        denom = jax.ops.segment_sum(a, node_batch_index, num_segments=B)
        a = a / (denom[node_batch_index] + 1e-16)
        r_ = jax.ops.segment_sum(a[:, None] * acc, node_batch_index, num_segments=B)
        q_star = jnp.concatenate([hl, r_], axis=1)
    pool = q_star[coupling_batch_index]
    nf = acc[coupling_index.reshape(-1)].reshape(C, -1)
    feats = jnp.concatenate([pool, nf, coupling_type.astype(jnp.float32)], axis=-1)
    stF = _col_stats(feats, 1000)
    fw1, fb1 = _fold(stF, C, p['fc_bn1_g'], p['fc_bn1_b'], p['fc_W1'].T, p['fc_b1'])
    zf, stZ = _lin_layer(feats, fw1, fb1, 1000)
    fw2, fb2 = _fold(stZ, C, p['fc_bn2_g'], p['fc_bn2_b'], p['fc_W2'].T, p['fc_b2'])
    preds = _lin_layer(zf, fw2, fb2, 1000, relu=False)[0]
    pred = jnp.take_along_axis(preds, coupling_type_back[:, None], axis=1).reshape(-1)
    return pred


# speedup vs baseline: 1.1558x; 1.1558x over previous
import functools

import jax
import jax.numpy as jnp
from jax import lax
from jax.experimental import pallas as pl
from jax.experimental.pallas import tpu as pltpu
from jax.experimental.pallas import tpu_sc as plsc

N = 10000
E = 160000
C = 20000
B = 64
H = 32
NF = 128
EF = 16
EH = 64
STEPS = 3

MSG_TILE = 2000


def _bn(x, g, b):
    m = jnp.mean(x, axis=0)
    v = jnp.var(x, axis=0)
    return g * (x - m) / jnp.sqrt(v + 1e-5) + b


def _msg_body(a_ref, e4_ref, w4t_ref, b4_ref, s_ref, out_ref):
    wt = jnp.dot(e4_ref[...], w4t_ref[...], preferred_element_type=jnp.float32)
    wt += b4_ref[0:1, :]
    a_rep = jnp.dot(a_ref[...], s_ref[...], preferred_element_type=jnp.float32)
    r = a_rep * wt
    r = r[:, :512] + r[:, 512:]
    r = r[:, :256] + r[:, 256:]
    r = r[:, :128] + r[:, 128:]
    r = r[:, :64] + r[:, 64:]
    out_ref[...] = r[:, :32] + r[:, 32:]


@functools.partial(jax.jit, static_argnames=("tile",))
def _msg_matmul(a_src, e4, w4t, b4row, sel, tile=MSG_TILE):
    return pl.pallas_call(
        _msg_body,
        grid=(E // tile,),
        in_specs=[
            pl.BlockSpec((tile, H), lambda i: (i, 0)),
            pl.BlockSpec((tile, EH), lambda i: (i, 0)),
            pl.BlockSpec((EH, H * H), lambda i: (0, 0)),
            pl.BlockSpec((8, H * H), lambda i: (0, 0)),
            pl.BlockSpec((H, H * H), lambda i: (0, 0)),
        ],
        out_specs=pl.BlockSpec((tile, H), lambda i: (i, 0)),
        out_shape=jax.ShapeDtypeStruct((E, H), jnp.float32),
    )(a_src, e4, w4t, b4row, sel)


_SC_NC = 2
_SC_NS = 16
_CH = 128
_NCHUNK = E // _CH
_CPC = _NCHUNK // _SC_NC
_STRIPE = 624
_TAIL = N - _STRIPE * _SC_NS


@jax.jit
def _sc_scatter_add(msg3, dst_flat, zstripe):
    mesh = plsc.VectorSubcoreMesh(core_axis_name="c", subcore_axis_name="s")

    @functools.partial(
        pl.kernel, mesh=mesh,
        out_type=jax.ShapeDtypeStruct((_SC_NC, N, H), jnp.float32),
        scratch_types=[
            pltpu.VMEM((_CH,), jnp.int32),
            pltpu.VMEM((_CH, H), jnp.float32),
            pltpu.VMEM_SHARED((N, H), jnp.float32),
        ],
        compiler_params=pltpu.CompilerParams(use_tc_tiling_on_sc=False),
    )
    def k(msg_hbm, dst_hbm, z_hbm, out_hbm, idx_v, rows_v, agg_sh):
        c = lax.axis_index("c")
        s = lax.axis_index("s")
        pltpu.sync_copy(z_hbm, agg_sh.at[pl.ds(s * _STRIPE, _STRIPE)])

        @pl.when(s == _SC_NS - 1)
        def _():
            pltpu.sync_copy(z_hbm.at[pl.ds(0, _TAIL)],
                            agg_sh.at[pl.ds(_STRIPE * _SC_NS, _TAIL)])
        plsc.subcore_barrier()

        def body(j, carry):
            cid_local = j * _SC_NS + s

            @pl.when(cid_local < _CPC)
            def _():
                cid = c * _CPC + cid_local
                pltpu.sync_copy(dst_hbm.at[pl.ds(cid * _CH, _CH)], idx_v)
                pltpu.sync_copy(msg_hbm.at[cid], rows_v)
                pltpu.sync_copy(rows_v, agg_sh.at[idx_v], add=True)
            return carry

        lax.fori_loop(0, (_CPC + _SC_NS - 1) // _SC_NS, body, 0)
        plsc.subcore_barrier()
        pltpu.sync_copy(agg_sh.at[pl.ds(s * _STRIPE, _STRIPE)],
                        out_hbm.at[c].at[pl.ds(s * _STRIPE, _STRIPE)])

        @pl.when(s == _SC_NS - 1)
        def _():
            pltpu.sync_copy(agg_sh.at[pl.ds(_STRIPE * _SC_NS, _TAIL)],
                            out_hbm.at[c].at[pl.ds(_STRIPE * _SC_NS, _TAIL)])

    return k(msg3, dst_flat, zstripe)


def _segment_sum_sc(msg, dst_flat, zstripe):
    parts = _sc_scatter_add(msg.reshape(_NCHUNK, _CH, H), dst_flat, zstripe)
    return parts[0] + parts[1]


def kernel(node, edge, edge_index, node_batch_index, coupling_index, coupling_type, coupling_type_back, coupling_value, coupling_batch_index, params):
    p = params
    ei = edge_index.T
    src = ei[0]
    dst = ei[1]
    x = jax.nn.relu(_bn(node, p['emb_bn1_g'], p['emb_bn1_b']) @ p['emb_W1'].T + p['emb_b1'])
    x = _bn(x, p['emb_bn2_g'], p['emb_bn2_b']) @ p['emb_W2'].T
    x = jax.nn.relu(x)
    h = x
    acc = x
    e = jax.nn.relu(_bn(edge, p['en_bn1_g'], p['en_bn1_b']) @ p['en_W1'].T + p['en_b1'])
    e = jax.nn.relu(_bn(e, p['en_bn2_g'], p['en_bn2_b']) @ p['en_W2'].T + p['en_b2'])
    e = jax.nn.relu(_bn(e, p['en_bn3_g'], p['en_bn3_b']) @ p['en_W3'].T + p['en_b3'])
    y3 = _bn(e, p['en_bn4_g'], p['en_bn4_b'])
    w4t = p['en_W4'].T
    b4row = jnp.broadcast_to(p['en_b4'][None, :], (8, H * H))
    counts = jnp.maximum(jax.ops.segment_sum(jnp.ones(E, jnp.float32), dst, num_segments=N), 1.0)[:, None]
    sel = jnp.repeat(jnp.eye(H, dtype=jnp.float32), H, axis=1)
    dst_flat = dst.astype(jnp.int32)
    zstripe = jnp.zeros((_STRIPE, H), jnp.float32)
    for _ in range(STEPS):
        a_src = acc[src]
        msg = _msg_matmul(a_src, y3, w4t, b4row, sel)
        agg = _segment_sum_sc(msg, dst_flat, zstripe) / counts
        m = jax.nn.relu(agg + acc @ p['conv_root'].T + p['conv_bias'])
        gi = m @ p['gru_W_ih'].T + p['gru_b_ih']
        gh = h @ p['gru_W_hh'].T + p['gru_b_hh']
        i_r, i_z, i_n = jnp.split(gi, 3, axis=1)
        h_r, h_z, h_n = jnp.split(gh, 3, axis=1)
        r = jax.nn.sigmoid(i_r + h_r)
        z = jax.nn.sigmoid(i_z + h_z)
        n = jnp.tanh(i_n + r * h_n)
        acc = (1.0 - z) * n + z * h
        h = acc
    q_star = jnp.zeros((B, 2 * H), jnp.float32)
    hl = jnp.zeros((B, H), jnp.float32)
    cl = jnp.zeros((B, H), jnp.float32)
    for _ in range(STEPS):
        gates = q_star @ p['lstm_W_ih'].T + p['lstm_b_ih'] + hl @ p['lstm_W_hh'].T + p['lstm_b_hh']
        gi_, gf_, gg_, go_ = jnp.split(gates, 4, axis=1)
        cl = jax.nn.sigmoid(gf_) * cl + jax.nn.sigmoid(gi_) * jnp.tanh(gg_)
        hl = jax.nn.sigmoid(go_) * jnp.tanh(cl)
        eatt = jnp.sum(acc * hl[node_batch_index], axis=-1)
        emax = jax.ops.segment_max(eatt, node_batch_index, num_segments=B)
        a = jnp.exp(eatt - emax[node_batch_index])
        denom = jax.ops.segment_sum(a, node_batch_index, num_segments=B)
        a = a / (denom[node_batch_index] + 1e-16)
        r_ = jax.ops.segment_sum(a[:, None] * acc, node_batch_index, num_segments=B)
        q_star = jnp.concatenate([hl, r_], axis=1)
    pool = q_star[coupling_batch_index]
    nf = acc[coupling_index.reshape(-1)].reshape(C, -1)
    feats = jnp.concatenate([pool, nf, coupling_type.astype(jnp.float32)], axis=-1)
    zf = jax.nn.relu(_bn(feats, p['fc_bn1_g'], p['fc_bn1_b']) @ p['fc_W1'].T + p['fc_b1'])
    preds = _bn(zf, p['fc_bn2_g'], p['fc_bn2_b']) @ p['fc_W2'].T + p['fc_b2']
    pred = jnp.take_along_axis(preds, coupling_type_back[:, None], axis=1).reshape(-1)
    return pred


# speedup vs baseline: 1.4458x; 1.2508x over previous
import functools

import jax
import jax.numpy as jnp
from jax import lax
from jax.experimental import pallas as pl
from jax.experimental.pallas import tpu as pltpu
from jax.experimental.pallas import tpu_sc as plsc

N = 10000
E = 160000
C = 20000
B = 64
H = 32
NF = 128
EF = 16
EH = 64
STEPS = 3

MSG_TILE = 2000


def _bn(x, g, b):
    m = jnp.mean(x, axis=0)
    v = jnp.var(x, axis=0)
    return g * (x - m) / jnp.sqrt(v + 1e-5) + b


def _msg_body(a_ref, e4_ref, w4t_ref, b4_ref, s_ref, out_ref):
    wt = jnp.dot(e4_ref[...], w4t_ref[...], preferred_element_type=jnp.float32)
    wt += b4_ref[0:1, :]
    a_rep = jnp.dot(a_ref[...], s_ref[...], preferred_element_type=jnp.float32)
    r = a_rep * wt
    r = r[:, :512] + r[:, 512:]
    r = r[:, :256] + r[:, 256:]
    r = r[:, :128] + r[:, 128:]
    r = r[:, :64] + r[:, 64:]
    out_ref[...] = r[:, :32] + r[:, 32:]


@functools.partial(jax.jit, static_argnames=("tile",))
def _msg_matmul(a_src, e4, w4t, b4row, sel, tile=MSG_TILE):
    return pl.pallas_call(
        _msg_body,
        grid=(E // tile,),
        in_specs=[
            pl.BlockSpec((tile, H), lambda i: (i, 0)),
            pl.BlockSpec((tile, EH), lambda i: (i, 0)),
            pl.BlockSpec((EH, H * H), lambda i: (0, 0)),
            pl.BlockSpec((8, H * H), lambda i: (0, 0)),
            pl.BlockSpec((H, H * H), lambda i: (0, 0)),
        ],
        out_specs=pl.BlockSpec((tile, H), lambda i: (i, 0)),
        out_shape=jax.ShapeDtypeStruct((E, H), jnp.float32),
    )(a_src, e4, w4t, b4row, sel)


_SC_NC = 2
_SC_NS = 16
_CH = 128
_NCHUNK = E // _CH
_CPC = _NCHUNK // _SC_NC
_STRIPE = 624
_TAIL = N - _STRIPE * _SC_NS


@jax.jit
def _sc_scatter_add(msg3, dst_flat, zstripe):
    mesh = plsc.VectorSubcoreMesh(core_axis_name="c", subcore_axis_name="s")

    @functools.partial(
        pl.kernel, mesh=mesh,
        out_type=jax.ShapeDtypeStruct((_SC_NC, N, H), jnp.float32),
        scratch_types=[
            pltpu.VMEM((_CH,), jnp.int32),
            pltpu.VMEM((_CH, H), jnp.float32),
            pltpu.VMEM_SHARED((N, H), jnp.float32),
        ],
        compiler_params=pltpu.CompilerParams(use_tc_tiling_on_sc=False),
    )
    def k(msg_hbm, dst_hbm, z_hbm, out_hbm, idx_v, rows_v, agg_sh):
        c = lax.axis_index("c")
        s = lax.axis_index("s")
        pltpu.sync_copy(z_hbm, agg_sh.at[pl.ds(s * _STRIPE, _STRIPE)])

        @pl.when(s == _SC_NS - 1)
        def _():
            pltpu.sync_copy(z_hbm.at[pl.ds(0, _TAIL)],
                            agg_sh.at[pl.ds(_STRIPE * _SC_NS, _TAIL)])
        plsc.subcore_barrier()

        def body(j, carry):
            cid_local = j * _SC_NS + s

            @pl.when(cid_local < _CPC)
            def _():
                cid = c * _CPC + cid_local
                pltpu.sync_copy(dst_hbm.at[pl.ds(cid * _CH, _CH)], idx_v)
                pltpu.sync_copy(msg_hbm.at[cid], rows_v)
                pltpu.sync_copy(rows_v, agg_sh.at[idx_v], add=True)
            return carry

        lax.fori_loop(0, (_CPC + _SC_NS - 1) // _SC_NS, body, 0)
        plsc.subcore_barrier()
        pltpu.sync_copy(agg_sh.at[pl.ds(s * _STRIPE, _STRIPE)],
                        out_hbm.at[c].at[pl.ds(s * _STRIPE, _STRIPE)])

        @pl.when(s == _SC_NS - 1)
        def _():
            pltpu.sync_copy(agg_sh.at[pl.ds(_STRIPE * _SC_NS, _TAIL)],
                            out_hbm.at[c].at[pl.ds(_STRIPE * _SC_NS, _TAIL)])

    return k(msg3, dst_flat, zstripe)


def _segment_sum_sc(msg, dst_flat, zstripe):
    parts = _sc_scatter_add(msg.reshape(_NCHUNK, _CH, H), dst_flat, zstripe)
    return parts[0] + parts[1]


def _s2s_body(acc_ref, nbi_ref, wih_ref, whh_ref, b_ref, out_ref):
    accv = acc_ref[...]
    nbi = nbi_ref[...]
    mask = (nbi == lax.broadcasted_iota(jnp.int32, (1, B), 1)).astype(jnp.float32)
    q = jnp.zeros((B, 2 * H), jnp.float32)
    hl = jnp.zeros((B, H), jnp.float32)
    cl = jnp.zeros((B, H), jnp.float32)
    for _ in range(STEPS):
        gates = (jnp.dot(q, wih_ref[...], preferred_element_type=jnp.float32)
                 + jnp.dot(hl, whh_ref[...], preferred_element_type=jnp.float32)
                 + b_ref[0:1, :])
        gi, gf = gates[:, :H], gates[:, H:2 * H]
        gg, go = gates[:, 2 * H:3 * H], gates[:, 3 * H:]
        cl = jax.nn.sigmoid(gf) * cl + jax.nn.sigmoid(gi) * jnp.tanh(gg)
        hl = jax.nn.sigmoid(go) * jnp.tanh(cl)
        hl_exp = jnp.dot(mask, hl, preferred_element_type=jnp.float32)
        ones_col = jnp.ones((H, 1), jnp.float32)
        eatt = jnp.dot(accv * hl_exp, ones_col, preferred_element_type=jnp.float32)
        emax = jnp.max(eatt * mask + (mask - 1.0) * 1e30, axis=0)[:, None]
        a = jnp.exp(eatt - jnp.dot(mask, emax, preferred_element_type=jnp.float32))
        denom = lax.dot_general(mask, a, (((0,), (0,)), ((), ())),
                                preferred_element_type=jnp.float32)
        a = a / (jnp.dot(mask, denom, preferred_element_type=jnp.float32) + 1e-16)
        r_ = lax.dot_general(mask, a * accv, (((0,), (0,)), ((), ())),
                             preferred_element_type=jnp.float32)
        q = jnp.concatenate([hl, r_], axis=1)
    out_ref[...] = q


@jax.jit
def _set2set(acc, nbi_col, wih, whh, brow):
    return pl.pallas_call(
        _s2s_body,
        in_specs=[
            pl.BlockSpec((N, H), lambda: (0, 0)),
            pl.BlockSpec((N, 1), lambda: (0, 0)),
            pl.BlockSpec((2 * H, 4 * H), lambda: (0, 0)),
            pl.BlockSpec((H, 4 * H), lambda: (0, 0)),
            pl.BlockSpec((8, 4 * H), lambda: (0, 0)),
        ],
        out_specs=pl.BlockSpec((B, 2 * H), lambda: (0, 0)),
        out_shape=jax.ShapeDtypeStruct((B, 2 * H), jnp.float32),
    )(acc, nbi_col, wih, whh, brow)


def kernel(node, edge, edge_index, node_batch_index, coupling_index, coupling_type, coupling_type_back, coupling_value, coupling_batch_index, params):
    p = params
    ei = edge_index.T
    src = ei[0]
    dst = ei[1]
    x = jax.nn.relu(_bn(node, p['emb_bn1_g'], p['emb_bn1_b']) @ p['emb_W1'].T + p['emb_b1'])
    x = _bn(x, p['emb_bn2_g'], p['emb_bn2_b']) @ p['emb_W2'].T
    x = jax.nn.relu(x)
    h = x
    acc = x
    e = jax.nn.relu(_bn(edge, p['en_bn1_g'], p['en_bn1_b']) @ p['en_W1'].T + p['en_b1'])
    e = jax.nn.relu(_bn(e, p['en_bn2_g'], p['en_bn2_b']) @ p['en_W2'].T + p['en_b2'])
    e = jax.nn.relu(_bn(e, p['en_bn3_g'], p['en_bn3_b']) @ p['en_W3'].T + p['en_b3'])
    y3 = _bn(e, p['en_bn4_g'], p['en_bn4_b'])
    w4t = p['en_W4'].T
    b4row = jnp.broadcast_to(p['en_b4'][None, :], (8, H * H))
    counts = jnp.maximum(jax.ops.segment_sum(jnp.ones(E, jnp.float32), dst, num_segments=N), 1.0)[:, None]
    sel = jnp.repeat(jnp.eye(H, dtype=jnp.float32), H, axis=1)
    dst_flat = dst.astype(jnp.int32)
    zstripe = jnp.zeros((_STRIPE, H), jnp.float32)
    for _ in range(STEPS):
        a_src = acc[src]
        msg = _msg_matmul(a_src, y3, w4t, b4row, sel)
        agg = _segment_sum_sc(msg, dst_flat, zstripe) / counts
        m = jax.nn.relu(agg + acc @ p['conv_root'].T + p['conv_bias'])
        gi = m @ p['gru_W_ih'].T + p['gru_b_ih']
        gh = h @ p['gru_W_hh'].T + p['gru_b_hh']
        i_r, i_z, i_n = jnp.split(gi, 3, axis=1)
        h_r, h_z, h_n = jnp.split(gh, 3, axis=1)
        r = jax.nn.sigmoid(i_r + h_r)
        z = jax.nn.sigmoid(i_z + h_z)
        n = jnp.tanh(i_n + r * h_n)
        acc = (1.0 - z) * n + z * h
        h = acc
    q_star = _set2set(
        acc,
        node_batch_index.astype(jnp.int32)[:, None],
        p['lstm_W_ih'].T,
        p['lstm_W_hh'].T,
        jnp.broadcast_to((p['lstm_b_ih'] + p['lstm_b_hh'])[None, :], (8, 4 * H)),
    )
    pool = q_star[coupling_batch_index]
    nf = acc[coupling_index.reshape(-1)].reshape(C, -1)
    feats = jnp.concatenate([pool, nf, coupling_type.astype(jnp.float32)], axis=-1)
    zf = jax.nn.relu(_bn(feats, p['fc_bn1_g'], p['fc_bn1_b']) @ p['fc_W1'].T + p['fc_b1'])
    preds = _bn(zf, p['fc_bn2_g'], p['fc_bn2_b']) @ p['fc_W2'].T + p['fc_b2']
    pred = jnp.take_along_axis(preds, coupling_type_back[:, None], axis=1).reshape(-1)
    return pred
